# baseline (speedup 1.0000x reference)
import numpy as np
import jax
import jax.numpy as jnp
from jax.experimental import pallas as pl
from jax.experimental.pallas import tpu as pltpu

_NUM_CLASSES = 1000
_B, _C, _W, _H = 64, 3, 224, 224
_ST = _B // 2
_HF = _ST // 2


def _constants():
    rng = np.random.default_rng(0)
    lam1 = float(rng.beta(0.2, 0.2))
    idx1 = rng.permutation(_HF)
    lam2 = float(rng.beta(1.0, 1.0))
    cx2 = int(rng.integers(_W))
    cy2 = int(rng.integers(_H))
    idx2 = rng.permutation(_ST - _HF)
    lam3 = float(rng.beta(0.2, 0.2))
    idx3 = rng.permutation(_HF)
    lam4 = float(rng.beta(1.0, 1.0))
    cx4 = int(rng.integers(_W))
    cy4 = int(rng.integers(_H))
    idx4 = rng.permutation((_B - _ST) - _HF)

    def cut_box(lam0, cx, cy):
        cut_rat = np.sqrt(1.0 - lam0)
        cut_w = int(_W * cut_rat)
        cut_h = int(_H * cut_rat)
        bbx1 = int(np.clip(cx - cut_w // 2, 0, _W))
        bby1 = int(np.clip(cy - cut_h // 2, 0, _H))
        bbx2 = int(np.clip(cx + cut_w // 2, 0, _W))
        bby2 = int(np.clip(cy + cut_h // 2, 0, _H))
        lam = 1.0 - (bbx2 - bbx1) * (bby2 - bby1) / float(_W * _H)
        return (bbx1, bby1, bbx2, bby2), lam

    box2, lame2 = cut_box(lam2, cx2, cy2)
    box4, lame4 = cut_box(lam4, cx4, cy4)

    lam_eff = np.array([lam1, lame2, lam3, lame4], np.float32)
    perm = np.concatenate(
        [idx1, _HF + idx2, _ST + idx3, _ST + _HF + idx4]
    ).astype(np.int32)
    return box2, box4, lam_eff, perm


_BOX2, _BOX4, _LAM_EFF, _PERM_NP = _constants()


_ROWS = 8

_BOXES = {1: _BOX2, 3: _BOX4}


def _body(a_ref, b_ref, lab_ref, img_ref, oi_ref, ol_ref, sem_ref):
    i = pl.program_id(0)
    iota = jax.lax.broadcasted_iota(jnp.int32, (1, _NUM_CLASSES), 1)
    steps_per_blk = _HF // _ROWS

    for blk in range(4):
        lam = float(_LAM_EFF[blk])

        @pl.when(i // steps_per_blk == blk)
        def _(blk=blk, lam=lam):
            if blk in _BOXES:
                bbx1, bby1, bbx2, bby2 = _BOXES[blk]
                copies = []
                for j in range(_ROWS):
                    c = pltpu.make_async_copy(
                        img_ref.at[a_ref[i * _ROWS + j]],
                        oi_ref.at[j],
                        sem_ref.at[j],
                    )
                    c.start()
                    copies.append(c)
                for c in copies:
                    c.wait()
                if bbx1 < bbx2 and bby1 < bby2:
                    for j in range(_ROWS):
                        xb = b_ref[i * _ROWS + j]
                        oi_ref[j, :, bbx1:bbx2, bby1:bby2] = (
                            img_ref[xb, :, bbx1:bbx2, bby1:bby2])
            else:
                for j in range(_ROWS):
                    xa = img_ref[a_ref[i * _ROWS + j]]
                    xb = img_ref[b_ref[i * _ROWS + j]]
                    oi_ref[j] = lam * xa + (1.0 - lam) * xb
            for j in range(_ROWS):
                la = lab_ref[a_ref[i * _ROWS + j]]
                lb = lab_ref[b_ref[i * _ROWS + j]]
                ol_ref[j] = (lam * (iota == la).astype(jnp.float32)
                             + (1.0 - lam) * (iota == lb).astype(jnp.float32))


def kernel(images, labels):
    key = (labels == (_NUM_CLASSES - 1)).astype(jnp.int32)
    order = jnp.argsort(key, stable=True).astype(jnp.int32)
    a_idx = order
    b_idx = order[jnp.asarray(_PERM_NP)]
    labels32 = labels.astype(jnp.int32)

    grid_spec = pltpu.PrefetchScalarGridSpec(
        num_scalar_prefetch=3,
        grid=(_B // _ROWS,),
        in_specs=[
            pl.BlockSpec((_B, _C, _W, _H), lambda i, a, b, l: (0, 0, 0, 0)),
        ],
        out_specs=[
            pl.BlockSpec((_ROWS, _C, _W, _H), lambda i, a, b, l: (i, 0, 0, 0)),
            pl.BlockSpec((_ROWS, 1, _NUM_CLASSES),
                         lambda i, a, b, l: (i, 0, 0)),
        ],
        scratch_shapes=[pltpu.SemaphoreType.DMA((_ROWS,))],
    )
    out_img, out_lab = pl.pallas_call(
        _body,
        grid_spec=grid_spec,
        out_shape=[
            jax.ShapeDtypeStruct((_B, _C, _W, _H), jnp.float32),
            jax.ShapeDtypeStruct((_B, 1, _NUM_CLASSES), jnp.float32),
        ],
        compiler_params=pltpu.CompilerParams(
            vmem_limit_bytes=100 * 1024 * 1024,
        ),
    )(a_idx, b_idx, labels32, images)
    return (out_img, out_lab.reshape(_B, _NUM_CLASSES))


# speedup vs baseline: 1.0955x; 1.0955x over previous
import numpy as np
import jax
import jax.numpy as jnp
from jax.experimental import pallas as pl
from jax.experimental.pallas import tpu as pltpu

_NUM_CLASSES = 1000
_B, _C, _W, _H = 64, 3, 224, 224
_ST = _B // 2
_HF = _ST // 2


def _constants():
    rng = np.random.default_rng(0)
    lam1 = float(rng.beta(0.2, 0.2))
    idx1 = rng.permutation(_HF)
    lam2 = float(rng.beta(1.0, 1.0))
    cx2 = int(rng.integers(_W))
    cy2 = int(rng.integers(_H))
    idx2 = rng.permutation(_ST - _HF)
    lam3 = float(rng.beta(0.2, 0.2))
    idx3 = rng.permutation(_HF)
    lam4 = float(rng.beta(1.0, 1.0))
    cx4 = int(rng.integers(_W))
    cy4 = int(rng.integers(_H))
    idx4 = rng.permutation((_B - _ST) - _HF)

    def cut_box(lam0, cx, cy):
        cut_rat = np.sqrt(1.0 - lam0)
        cut_w = int(_W * cut_rat)
        cut_h = int(_H * cut_rat)
        bbx1 = int(np.clip(cx - cut_w // 2, 0, _W))
        bby1 = int(np.clip(cy - cut_h // 2, 0, _H))
        bbx2 = int(np.clip(cx + cut_w // 2, 0, _W))
        bby2 = int(np.clip(cy + cut_h // 2, 0, _H))
        lam = 1.0 - (bbx2 - bbx1) * (bby2 - bby1) / float(_W * _H)
        return (bbx1, bby1, bbx2, bby2), lam

    box2, lame2 = cut_box(lam2, cx2, cy2)
    box4, lame4 = cut_box(lam4, cx4, cy4)

    lam_eff = np.array([lam1, lame2, lam3, lame4], np.float32)
    perm = np.concatenate(
        [idx1, _HF + idx2, _ST + idx3, _ST + _HF + idx4]
    ).astype(np.int32)
    return box2, box4, lam_eff, perm


_BOX2, _BOX4, _LAM_EFF, _PERM_NP = _constants()


_ROWS = 8

_BOXES = {1: _BOX2, 3: _BOX4}


def _body(a_ref, b_ref, lab_ref, img_ref, oi_ref, ol_ref):
    i = pl.program_id(0)
    iota = jax.lax.broadcasted_iota(jnp.int32, (1, _NUM_CLASSES), 1)
    steps_per_blk = _HF // _ROWS

    for blk in range(4):
        lam = float(_LAM_EFF[blk])

        @pl.when(i // steps_per_blk == blk)
        def _(blk=blk, lam=lam):
            if blk in _BOXES:
                bbx1, bby1, bbx2, bby2 = _BOXES[blk]
                a_rects = [
                    (0, bbx1, 0, _H),
                    (bbx2, _W, 0, _H),
                    (bbx1, bbx2, 0, bby1),
                    (bbx1, bbx2, bby2, _H),
                ]
                a_rects = [(x1, x2, y1, y2) for x1, x2, y1, y2 in a_rects
                           if x1 < x2 and y1 < y2]
                for j in range(_ROWS):
                    xa = a_ref[i * _ROWS + j]
                    xb = b_ref[i * _ROWS + j]
                    for x1, x2, y1, y2 in a_rects:
                        oi_ref[j, :, x1:x2, y1:y2] = (
                            img_ref[xa, :, x1:x2, y1:y2])
                    if bbx1 < bbx2 and bby1 < bby2:
                        oi_ref[j, :, bbx1:bbx2, bby1:bby2] = (
                            img_ref[xb, :, bbx1:bbx2, bby1:bby2])
            else:
                for j in range(_ROWS):
                    xa = img_ref[a_ref[i * _ROWS + j]]
                    xb = img_ref[b_ref[i * _ROWS + j]]
                    oi_ref[j] = lam * xa + (1.0 - lam) * xb
            for j in range(_ROWS):
                la = lab_ref[a_ref[i * _ROWS + j]]
                lb = lab_ref[b_ref[i * _ROWS + j]]
                ol_ref[j] = (lam * (iota == la).astype(jnp.float32)
                             + (1.0 - lam) * (iota == lb).astype(jnp.float32))


def kernel(images, labels):
    key = (labels == (_NUM_CLASSES - 1)).astype(jnp.int32)
    order = jnp.argsort(key, stable=True).astype(jnp.int32)
    a_idx = order
    b_idx = order[jnp.asarray(_PERM_NP)]
    labels32 = labels.astype(jnp.int32)

    grid_spec = pltpu.PrefetchScalarGridSpec(
        num_scalar_prefetch=3,
        grid=(_B // _ROWS,),
        in_specs=[
            pl.BlockSpec((_B, _C, _W, _H), lambda i, a, b, l: (0, 0, 0, 0)),
        ],
        out_specs=[
            pl.BlockSpec((_ROWS, _C, _W, _H), lambda i, a, b, l: (i, 0, 0, 0)),
            pl.BlockSpec((_ROWS, 1, _NUM_CLASSES),
                         lambda i, a, b, l: (i, 0, 0)),
        ],
    )
    out_img, out_lab = pl.pallas_call(
        _body,
        grid_spec=grid_spec,
        out_shape=[
            jax.ShapeDtypeStruct((_B, _C, _W, _H), jnp.float32),
            jax.ShapeDtypeStruct((_B, 1, _NUM_CLASSES), jnp.float32),
        ],
        compiler_params=pltpu.CompilerParams(
            vmem_limit_bytes=100 * 1024 * 1024,
        ),
    )(a_idx, b_idx, labels32, images)
    return (out_img, out_lab.reshape(_B, _NUM_CLASSES))


# speedup vs baseline: 1.1546x; 1.0540x over previous
import numpy as np
import jax
import jax.numpy as jnp
from jax.experimental import pallas as pl
from jax.experimental.pallas import tpu as pltpu

_NUM_CLASSES = 1000
_B, _C, _W, _H = 64, 3, 224, 224
_ST = _B // 2
_HF = _ST // 2


def _constants():
    rng = np.random.default_rng(0)
    lam1 = float(rng.beta(0.2, 0.2))
    idx1 = rng.permutation(_HF)
    lam2 = float(rng.beta(1.0, 1.0))
    cx2 = int(rng.integers(_W))
    cy2 = int(rng.integers(_H))
    idx2 = rng.permutation(_ST - _HF)
    lam3 = float(rng.beta(0.2, 0.2))
    idx3 = rng.permutation(_HF)
    lam4 = float(rng.beta(1.0, 1.0))
    cx4 = int(rng.integers(_W))
    cy4 = int(rng.integers(_H))
    idx4 = rng.permutation((_B - _ST) - _HF)

    def cut_box(lam0, cx, cy):
        cut_rat = np.sqrt(1.0 - lam0)
        cut_w = int(_W * cut_rat)
        cut_h = int(_H * cut_rat)
        bbx1 = int(np.clip(cx - cut_w // 2, 0, _W))
        bby1 = int(np.clip(cy - cut_h // 2, 0, _H))
        bbx2 = int(np.clip(cx + cut_w // 2, 0, _W))
        bby2 = int(np.clip(cy + cut_h // 2, 0, _H))
        lam = 1.0 - (bbx2 - bbx1) * (bby2 - bby1) / float(_W * _H)
        return (bbx1, bby1, bbx2, bby2), lam

    box2, lame2 = cut_box(lam2, cx2, cy2)
    box4, lame4 = cut_box(lam4, cx4, cy4)

    lam_eff = np.array([lam1, lame2, lam3, lame4], np.float32)
    perms = [idx1.astype(np.int32), idx2.astype(np.int32),
             idx3.astype(np.int32), idx4.astype(np.int32)]
    return box2, box4, lam_eff, perms


_BOX2, _BOX4, _LAM_EFF, _PERMS = _constants()
_BOXES = {1: _BOX2, 3: _BOX4}
_PERM_FULL = np.concatenate(
    [blk * _HF + p for blk, p in enumerate(_PERMS)]).astype(np.int32)


def _body(a_ref, b_ref, lab_ref, *refs):
    xs = refs[:_HF]
    oi_ref, ol_ref = refs[_HF], refs[_HF + 1]
    i = pl.program_id(0)
    iota = jax.lax.broadcasted_iota(jnp.int32, (1, _NUM_CLASSES), 1)

    for blk in range(4):
        lam = float(_LAM_EFF[blk])
        ploc = [int(v) for v in _PERMS[blk]]

        @pl.when(i == blk)
        def _(blk=blk, lam=lam, ploc=ploc):
            if blk in _BOXES:
                bbx1, bby1, bbx2, bby2 = _BOXES[blk]
                a_rects = [
                    (0, bbx1, 0, _H),
                    (bbx2, _W, 0, _H),
                    (bbx1, bbx2, 0, bby1),
                    (bbx1, bbx2, bby2, _H),
                ]
                a_rects = [r for r in a_rects if r[0] < r[1] and r[2] < r[3]]
                for j in range(_HF):
                    for x1, x2, y1, y2 in a_rects:
                        oi_ref[j, :, x1:x2, y1:y2] = xs[j][0, :, x1:x2, y1:y2]
                    if bbx1 < bbx2 and bby1 < bby2:
                        oi_ref[j, :, bbx1:bbx2, bby1:bby2] = (
                            xs[ploc[j]][0, :, bbx1:bbx2, bby1:bby2])
            else:
                for j in range(_HF):
                    oi_ref[j] = (lam * xs[j][0]
                                 + (1.0 - lam) * xs[ploc[j]][0])
            for j in range(_HF):
                la = lab_ref[a_ref[i * _HF + j]]
                lb = lab_ref[b_ref[i * _HF + j]]
                ol_ref[j] = (lam * (iota == la).astype(jnp.float32)
                             + (1.0 - lam) * (iota == lb).astype(jnp.float32))


def kernel(images, labels):
    key = (labels == (_NUM_CLASSES - 1)).astype(jnp.int32)
    order = jnp.argsort(key, stable=True).astype(jnp.int32)
    a_idx = order
    b_idx = order[jnp.asarray(_PERM_FULL)]
    labels32 = labels.astype(jnp.int32)

    def a_map(j):
        return lambda i, a, b, l: (a[i * _HF + j], 0, 0, 0)

    grid_spec = pltpu.PrefetchScalarGridSpec(
        num_scalar_prefetch=3,
        grid=(_B // _HF,),
        in_specs=[pl.BlockSpec((1, _C, _W, _H), a_map(j))
                  for j in range(_HF)],
        out_specs=[
            pl.BlockSpec((_HF, _C, _W, _H), lambda i, a, b, l: (i, 0, 0, 0)),
            pl.BlockSpec((_HF, 1, _NUM_CLASSES), lambda i, a, b, l: (i, 0, 0)),
        ],
    )
    out_img, out_lab = pl.pallas_call(
        _body,
        grid_spec=grid_spec,
        out_shape=[
            jax.ShapeDtypeStruct((_B, _C, _W, _H), jnp.float32),
            jax.ShapeDtypeStruct((_B, 1, _NUM_CLASSES), jnp.float32),
        ],
        compiler_params=pltpu.CompilerParams(
            vmem_limit_bytes=100 * 1024 * 1024,
        ),
    )(a_idx, b_idx, labels32, *([images] * _HF))
    return (out_img, out_lab.reshape(_B, _NUM_CLASSES))
